# Initial kernel scaffold; baseline (speedup 1.0000x reference)
import jax
import jax.numpy as jnp
from jax.experimental import pallas as pl


def kernel(source, target, proj):
    raise NotImplementedError("write your pallas kernel here")


# speedup vs baseline: 1.9594x; 1.9594x over previous
import jax
import jax.numpy as jnp
from jax.experimental import pallas as pl
from jax.experimental.pallas import tpu as pltpu

_CB = 128
_TN = 2048


def _matmul_body(src_ref, tgt_ref, p_ref, xp_ref, yp_ref):
    pb = p_ref[...]
    norm = jnp.sqrt(jnp.sum(pb * pb, axis=0, keepdims=True))
    pb = pb / jnp.maximum(norm, 1e-30)
    xp_ref[...] = jnp.dot(src_ref[...], pb, preferred_element_type=jnp.float32)
    yp_ref[...] = jnp.dot(tgt_ref[...], pb, preferred_element_type=jnp.float32)


_C = 256


def _bitonic_inplace2(xs, ys, n):
    lgn = n.bit_length() - 1
    lgc = _C.bit_length() - 1
    nstages = lgn * (lgn + 1) // 2
    nch = n // _C
    li = jax.lax.broadcasted_iota(jnp.int32, (_C, 1), 0)

    def stage(_, carry):
        lk, lj = carry
        j = 1 << lj

        def pair_pass():
            b = lj - lgc

            def body(m, __):
                qa = ((m >> b) << (b + 1)) | (m & ((1 << b) - 1))
                sa = qa * _C
                sb = sa + (1 << lj)
                asc = ((sa >> lk) & 1) == 0
                for buf in (xs, ys):
                    a = buf[pl.ds(sa, _C), :]
                    c = buf[pl.ds(sb, _C), :]
                    mn, mx = jnp.minimum(a, c), jnp.maximum(a, c)
                    buf[pl.ds(sa, _C), :] = jnp.where(asc, mn, mx)
                    buf[pl.ds(sb, _C), :] = jnp.where(asc, mx, mn)
                return 0

            jax.lax.fori_loop(0, nch // 2, body, 0)

        def intra_pass():
            def body(q, __):
                s = q * _C
                gri = li + s
                bj = (gri >> lj) & 1
                take_min = ((gri >> lk) & 1) == bj
                sel_up = bj == 1
                for buf in (xs, ys):
                    a = buf[pl.ds(s, _C), :]
                    pa = jnp.where(sel_up, pltpu.roll(a, j, 0), pltpu.roll(a, _C - j, 0))
                    buf[pl.ds(s, _C), :] = jnp.where(
                        take_min, jnp.minimum(a, pa), jnp.maximum(a, pa)
                    )
                return 0

            jax.lax.fori_loop(0, nch, body, 0)

        jax.lax.cond(lj >= lgc, pair_pass, intra_pass)
        done = lj == 0
        return jnp.where(done, lk + 1, lk), jnp.where(done, lk, lj - 1)

    jax.lax.fori_loop(0, nstages, stage, (jnp.int32(1), jnp.int32(0)))


def _sort_body(xp_hbm, yp_hbm, out_ref, xs, ys, semx, semy):
    c = pl.program_id(0)
    n = xs.shape[0]
    cpx = pltpu.make_async_copy(xp_hbm.at[:, pl.ds(c * _CB, _CB)], xs, semx)
    cpy = pltpu.make_async_copy(yp_hbm.at[:, pl.ds(c * _CB, _CB)], ys, semy)
    cpx.start()
    cpy.start()
    cpx.wait()
    cpy.wait()
    _bitonic_inplace2(xs, ys, n)
    d = xs[...] - ys[...]
    s = jnp.sum(d * d, axis=0, keepdims=True)
    out_ref[...] = jnp.broadcast_to(s, out_ref.shape)


def kernel(source, target, proj):
    n, d = source.shape
    nproj = proj.shape[1]
    ncb = -(-nproj // _CB)
    cp = ncb * _CB
    projp = jnp.pad(proj, ((0, 0), (0, cp - nproj)))

    nt = n // _TN
    xp, yp = pl.pallas_call(
        _matmul_body,
        grid=(nt, ncb),
        in_specs=[
            pl.BlockSpec((_TN, d), lambda i, c: (i, 0)),
            pl.BlockSpec((_TN, d), lambda i, c: (i, 0)),
            pl.BlockSpec((d, _CB), lambda i, c: (0, c)),
        ],
        out_specs=[
            pl.BlockSpec((_TN, _CB), lambda i, c: (i, c)),
            pl.BlockSpec((_TN, _CB), lambda i, c: (i, c)),
        ],
        out_shape=[
            jax.ShapeDtypeStruct((n, cp), jnp.float32),
            jax.ShapeDtypeStruct((n, cp), jnp.float32),
        ],
    )(source, target, projp)

    sums = pl.pallas_call(
        _sort_body,
        grid=(ncb,),
        in_specs=[
            pl.BlockSpec(memory_space=pl.ANY),
            pl.BlockSpec(memory_space=pl.ANY),
        ],
        out_specs=pl.BlockSpec((1, 8, _CB), lambda c: (c, 0, 0)),
        out_shape=jax.ShapeDtypeStruct((ncb, 8, _CB), jnp.float32),
        scratch_shapes=[
            pltpu.VMEM((n, _CB), jnp.float32),
            pltpu.VMEM((n, _CB), jnp.float32),
            pltpu.SemaphoreType.DMA,
            pltpu.SemaphoreType.DMA,
        ],
    )(xp, yp)

    flat = sums[:, 0, :].reshape(-1)[:nproj]
    return jnp.sqrt(jnp.sum(flat) / (n * nproj))
